# Initial kernel scaffold; baseline (speedup 1.0000x reference)
import jax
import jax.numpy as jnp
from jax.experimental import pallas as pl


def kernel(x, edge_index, batch, W_enc, b_enc, W_convs, b_convs, W_proj, b_proj, ln_g, ln_b):
    raise NotImplementedError("write your pallas kernel here")


# speedup vs baseline: 16.1661x; 16.1661x over previous
import functools

import jax
import jax.numpy as jnp
from jax import lax
from jax.experimental import pallas as pl
from jax.experimental.pallas import tpu as pltpu
from jax.experimental.pallas import tpu_sc as plsc

N = 10000
NPAD = 10240
E = 320000
ATOM_DIM = 128
HIDDEN = 128
NODE_DIM = 64
NUM_GRAPHS = 64

NC, NS = 2, 16
NW = NC * NS
EPW = E // NW
K = 80
NCH = EPW // K
RPS = NPAD // NS
DEGW = 128

_sc_mesh = plsc.VectorSubcoreMesh(core_axis_name="c", subcore_axis_name="s")


@functools.partial(
    pl.kernel,
    mesh=_sc_mesh,
    out_type=jax.ShapeDtypeStruct((NC, NPAD, DEGW), jnp.float32),
    scratch_types=[
        pltpu.VMEM_SHARED((NPAD, DEGW), jnp.float32),
        pltpu.VMEM((K,), jnp.int32),
        pltpu.VMEM((K, DEGW), jnp.float32),
    ],
)
def _deg_sc(dst_hbm, ones_hbm, zero_hbm, out_hbm, acc, idx, ones_v):
    cid = lax.axis_index("c")
    sid = lax.axis_index("s")
    wid = sid * NC + cid
    pltpu.sync_copy(zero_hbm.at[pl.ds(sid * RPS, RPS)],
                    acc.at[pl.ds(sid * RPS, RPS)])
    pltpu.sync_copy(ones_hbm, ones_v)
    plsc.subcore_barrier()
    ebase = wid * EPW
    pltpu.sync_copy(dst_hbm.at[pl.ds(ebase, K)], idx)

    def body(c, carry):
        pltpu.sync_copy(ones_v, acc.at[idx], add=True)
        cn = jnp.minimum(c + 1, NCH - 1)
        pltpu.sync_copy(dst_hbm.at[pl.ds(ebase + cn * K, K)], idx)
        return carry

    lax.fori_loop(0, NCH, body, 0)
    plsc.subcore_barrier()
    pltpu.sync_copy(acc.at[pl.ds(sid * RPS, RPS)],
                    out_hbm.at[cid, pl.ds(sid * RPS, RPS)])


@functools.partial(
    pl.kernel,
    mesh=_sc_mesh,
    out_type=jax.ShapeDtypeStruct((NC, NPAD, HIDDEN), jnp.float32),
    scratch_types=[
        pltpu.VMEM_SHARED((NPAD, HIDDEN), jnp.float32),
        pltpu.VMEM((K,), jnp.int32),
        pltpu.VMEM((K,), jnp.int32),
        pltpu.VMEM((K,), jnp.int32),
        pltpu.VMEM((K,), jnp.int32),
        pltpu.VMEM((K, HIDDEN), jnp.float32),
        pltpu.VMEM((K, HIDDEN), jnp.float32),
        pltpu.SemaphoreType.DMA,
        pltpu.SemaphoreType.DMA,
    ],
)
def _agg_sc(g_hbm, src_hbm, dst_hbm, zero_hbm, out_hbm,
            acc, is0, is1, id0, id1, r0, r1, sem0, sem1):
    cid = lax.axis_index("c")
    sid = lax.axis_index("s")
    wid = sid * NC + cid
    iss = [is0, is1]
    ids = [id0, id1]
    rows = [r0, r1]
    sems = [sem0, sem1]
    ebase = wid * EPW

    pltpu.sync_copy(zero_hbm.at[pl.ds(sid * RPS, RPS)],
                    acc.at[pl.ds(sid * RPS, RPS)])
    plsc.subcore_barrier()

    def load_idx(c, b):
        off = ebase + c * K
        pltpu.sync_copy(src_hbm.at[pl.ds(off, K)], iss[b])
        pltpu.sync_copy(dst_hbm.at[pl.ds(off, K)], ids[b])

    for b in range(2):
        load_idx(b, b)
        pltpu.async_copy(g_hbm.at[iss[b]], rows[b], sems[b])

    def body(i, carry):
        c0 = i * 2
        for b in range(2):
            c = c0 + b
            pltpu.make_async_copy(g_hbm.at[iss[b]], rows[b], sems[b]).wait()
            pltpu.sync_copy(rows[b], acc.at[ids[b]], add=True)
            cn = jnp.minimum(c + 2, NCH - 1)
            load_idx(cn, b)
            pltpu.async_copy(g_hbm.at[iss[b]], rows[b], sems[b])
        return carry

    lax.fori_loop(0, NCH // 2, body, 0)
    pltpu.make_async_copy(g_hbm.at[iss[0]], rows[0], sems[0]).wait()
    pltpu.sync_copy(rows[0], acc.at[ids[0]], add=True)
    pltpu.make_async_copy(g_hbm.at[iss[1]], rows[1], sems[1]).wait()
    plsc.subcore_barrier()
    pltpu.sync_copy(acc.at[pl.ds(sid * RPS, RPS)],
                    out_hbm.at[cid, pl.ds(sid * RPS, RPS)])


_RB = 1000
_GRID = N // _RB


def _dis_of(degp):
    d = 1.0 + degp[0, :, 0:1] + degp[1, :, 0:1]
    return lax.rsqrt(d)


def _encode_body(x_ref, we_ref, be_ref, w1_ref, degp_ref, g1_ref):
    h = jnp.dot(x_ref[...], we_ref[...], preferred_element_type=jnp.float32)
    h = h + be_ref[...]
    dis = _dis_of(degp_ref[...])
    g1_ref[...] = dis * jnp.dot(h, w1_ref[...], preferred_element_type=jnp.float32)


def _encode_tc(x, W_enc, b_enc2, W1, degp):
    return pl.pallas_call(
        _encode_body,
        grid=(_GRID,),
        in_specs=[
            pl.BlockSpec((_RB, ATOM_DIM), lambda i: (i, 0)),
            pl.BlockSpec((ATOM_DIM, HIDDEN), lambda i: (0, 0)),
            pl.BlockSpec((1, HIDDEN), lambda i: (0, 0)),
            pl.BlockSpec((HIDDEN, HIDDEN), lambda i: (0, 0)),
            pl.BlockSpec((NC, _RB, DEGW), lambda i: (0, i, 0)),
        ],
        out_specs=pl.BlockSpec((_RB, HIDDEN), lambda i: (i, 0)),
        out_shape=jax.ShapeDtypeStruct((N, HIDDEN), jnp.float32),
    )(x, W_enc, b_enc2, W1, degp)


def _layer_body(p_ref, g_ref, degp_ref, b_ref, w_ref, out_ref):
    dis = _dis_of(degp_ref[...])
    agg = p_ref[0] + p_ref[1] + g_ref[...]
    h = jnp.maximum(dis * agg + b_ref[...], 0.0)
    out_ref[...] = dis * jnp.dot(h, w_ref[...], preferred_element_type=jnp.float32)


def _layer_tc(p, g, degp, b2, Wn):
    return pl.pallas_call(
        _layer_body,
        grid=(_GRID,),
        in_specs=[
            pl.BlockSpec((NC, _RB, HIDDEN), lambda i: (0, i, 0)),
            pl.BlockSpec((_RB, HIDDEN), lambda i: (i, 0)),
            pl.BlockSpec((NC, _RB, DEGW), lambda i: (0, i, 0)),
            pl.BlockSpec((1, HIDDEN), lambda i: (0, 0)),
            pl.BlockSpec((HIDDEN, HIDDEN), lambda i: (0, 0)),
        ],
        out_specs=pl.BlockSpec((_RB, HIDDEN), lambda i: (i, 0)),
        out_shape=jax.ShapeDtypeStruct((N, HIDDEN), jnp.float32),
    )(p, g, degp, b2, Wn)


def _pool_body(p_ref, g_ref, degp_ref, b_ref, batch_ref, sums_ref, cnt_ref):
    i = pl.program_id(0)
    dis = _dis_of(degp_ref[...])
    agg = p_ref[0] + p_ref[1] + g_ref[...]
    h = jnp.maximum(dis * agg + b_ref[...], 0.0)
    gids = lax.broadcasted_iota(jnp.int32, (NUM_GRAPHS, _RB), 0)
    onehot = (gids == batch_ref[0]).astype(jnp.float32)
    psum = jnp.dot(onehot, h, preferred_element_type=jnp.float32)
    pcnt = jnp.sum(onehot, axis=1, keepdims=True)
    pcnt = jnp.broadcast_to(pcnt, (NUM_GRAPHS, HIDDEN))

    @pl.when(i == 0)
    def _():
        sums_ref[...] = jnp.zeros_like(sums_ref)
        cnt_ref[...] = jnp.zeros_like(cnt_ref)

    sums_ref[...] += psum
    cnt_ref[...] += pcnt


def _pool_tc(p, g, degp, b2, batch3):
    return pl.pallas_call(
        _pool_body,
        grid=(_GRID,),
        in_specs=[
            pl.BlockSpec((NC, _RB, HIDDEN), lambda i: (0, i, 0)),
            pl.BlockSpec((_RB, HIDDEN), lambda i: (i, 0)),
            pl.BlockSpec((NC, _RB, DEGW), lambda i: (0, i, 0)),
            pl.BlockSpec((1, HIDDEN), lambda i: (0, 0)),
            pl.BlockSpec((1, 1, _RB), lambda i: (i, 0, 0)),
        ],
        out_specs=[
            pl.BlockSpec((NUM_GRAPHS, HIDDEN), lambda i: (0, 0)),
            pl.BlockSpec((NUM_GRAPHS, HIDDEN), lambda i: (0, 0)),
        ],
        out_shape=[
            jax.ShapeDtypeStruct((NUM_GRAPHS, HIDDEN), jnp.float32),
            jax.ShapeDtypeStruct((NUM_GRAPHS, HIDDEN), jnp.float32),
        ],
    )(p, g, degp, b2, batch3)


def _head_body(sums_ref, cnt_ref, wp_ref, bp_ref, lng_ref, lnb_ref, out_ref):
    mol = sums_ref[...] / jnp.maximum(cnt_ref[...], 1.0)
    y = jnp.dot(mol, wp_ref[...], preferred_element_type=jnp.float32)
    y = y + bp_ref[...]
    mu = jnp.mean(y, axis=1, keepdims=True)
    var = jnp.mean((y - mu) * (y - mu), axis=1, keepdims=True)
    y = (y - mu) * lax.rsqrt(var + 1e-5)
    out_ref[...] = y * lng_ref[...] + lnb_ref[...]


def _head_tc(sums, cnt, W_proj, bp2, lng2, lnb2):
    return pl.pallas_call(
        _head_body,
        out_shape=jax.ShapeDtypeStruct((NUM_GRAPHS, NODE_DIM), jnp.float32),
    )(sums, cnt, W_proj, bp2, lng2, lnb2)


def kernel(x, edge_index, batch, W_enc, b_enc, W_convs, b_convs,
           W_proj, b_proj, ln_g, ln_b):
    src = edge_index[0]
    dst = edge_index[1]
    batch3 = batch.reshape(_GRID, 1, _RB)
    zeros_h = jnp.zeros((NPAD, HIDDEN), jnp.float32)
    ones_k = jnp.ones((K, DEGW), jnp.float32)
    b_enc2 = b_enc.reshape(1, HIDDEN)
    bc2 = [b_convs[i].reshape(1, HIDDEN) for i in range(3)]

    degp = _deg_sc(dst, ones_k, zeros_h)
    g = _encode_tc(x, W_enc, b_enc2, W_convs[0], degp)
    p = _agg_sc(g, src, dst, zeros_h)
    g = _layer_tc(p, g, degp, bc2[0], W_convs[1])
    p = _agg_sc(g, src, dst, zeros_h)
    g = _layer_tc(p, g, degp, bc2[1], W_convs[2])
    p = _agg_sc(g, src, dst, zeros_h)
    sums, cnt = _pool_tc(p, g, degp, bc2[2], batch3)
    return _head_tc(sums, cnt, W_proj, b_proj.reshape(1, NODE_DIM),
                    ln_g.reshape(1, NODE_DIM), ln_b.reshape(1, NODE_DIM))


# speedup vs baseline: 23.5884x; 1.4591x over previous
import functools

import jax
import jax.numpy as jnp
from jax import lax
from jax.experimental import pallas as pl
from jax.experimental.pallas import tpu as pltpu
from jax.experimental.pallas import tpu_sc as plsc

N = 10000
NPAD = 10240
E = 320000
ATOM_DIM = 128
HIDDEN = 128
NODE_DIM = 64
NUM_GRAPHS = 64

NC, NS = 2, 16
NW = NC * NS
EPW = E // NW
K = 80
NCH = EPW // K
RPS = NPAD // NS
DEGW = 128

_sc_mesh = plsc.VectorSubcoreMesh(core_axis_name="c", subcore_axis_name="s")


@functools.partial(
    pl.kernel,
    mesh=_sc_mesh,
    out_type=jax.ShapeDtypeStruct((NC, NPAD, DEGW), jnp.float32),
    scratch_types=[
        pltpu.VMEM_SHARED((NPAD, DEGW), jnp.float32),
        pltpu.VMEM((NCH, K), jnp.int32),
        pltpu.VMEM((K, DEGW), jnp.float32),
        pltpu.SemaphoreType.DMA,
        pltpu.SemaphoreType.DMA,
    ],
)
def _deg_sc(dst3_hbm, ones_hbm, zero_hbm, out_hbm, acc, dsts, ones_v, sem0, sem1):
    cid = lax.axis_index("c")
    sid = lax.axis_index("s")
    wid = sid * NC + cid
    sems = [sem0, sem1]
    pltpu.sync_copy(zero_hbm.at[pl.ds(sid * RPS, RPS)],
                    acc.at[pl.ds(sid * RPS, RPS)])
    pltpu.sync_copy(ones_hbm, ones_v)
    pltpu.sync_copy(dst3_hbm.at[wid], dsts)
    plsc.subcore_barrier()

    for b in range(2):
        pltpu.async_copy(ones_v, acc.at[dsts.at[b]], sems[b], add=True)

    def body(i, carry):
        c0 = i * 2
        for b in range(2):
            c = c0 + b
            pltpu.make_async_copy(ones_v, acc.at[dsts.at[c]], sems[b]).wait()
            cn = c + 2

            @pl.when(cn <= NCH - 1)
            def _():
                pltpu.async_copy(ones_v, acc.at[dsts.at[cn]], sems[b], add=True)

        return carry

    lax.fori_loop(0, NCH // 2, body, 0)
    if NCH % 2:
        pltpu.make_async_copy(ones_v, acc.at[dsts.at[NCH - 1]], sems[0]).wait()
    plsc.subcore_barrier()
    pltpu.sync_copy(acc.at[pl.ds(sid * RPS, RPS)],
                    out_hbm.at[cid, pl.ds(sid * RPS, RPS)])


@functools.partial(
    pl.kernel,
    mesh=_sc_mesh,
    out_type=jax.ShapeDtypeStruct((NC, NPAD, HIDDEN), jnp.float32),
    scratch_types=[
        pltpu.VMEM_SHARED((NPAD, HIDDEN), jnp.float32),
        pltpu.VMEM((K,), jnp.int32),
        pltpu.VMEM((K,), jnp.int32),
        pltpu.VMEM((K,), jnp.int32),
        pltpu.VMEM((K,), jnp.int32),
        pltpu.VMEM((K, HIDDEN), jnp.float32),
        pltpu.VMEM((K, HIDDEN), jnp.float32),
        pltpu.SemaphoreType.DMA, pltpu.SemaphoreType.DMA,
        pltpu.SemaphoreType.DMA, pltpu.SemaphoreType.DMA,
        pltpu.SemaphoreType.DMA, pltpu.SemaphoreType.DMA,
    ],
)
def _agg_sc(g_hbm, src_hbm, dst_hbm, zero_hbm, out_hbm,
            acc, is0, is1, id0, id1, r0, r1,
            gsem0, gsem1, ssem0, ssem1, dsem0, dsem1):
    cid = lax.axis_index("c")
    sid = lax.axis_index("s")
    wid = sid * NC + cid
    iss = [is0, is1]
    ids = [id0, id1]
    rows = [r0, r1]
    gsems = [gsem0, gsem1]
    isems = [ssem0, ssem1]
    dsems = [dsem0, dsem1]
    ebase = wid * EPW

    pltpu.sync_copy(zero_hbm.at[pl.ds(sid * RPS, RPS)],
                    acc.at[pl.ds(sid * RPS, RPS)])
    plsc.subcore_barrier()

    def issue_src(c, b):
        pltpu.async_copy(src_hbm.at[pl.ds(ebase + c * K, K)], iss[b], isems[b])

    def issue_dst(c, b):
        pltpu.async_copy(dst_hbm.at[pl.ds(ebase + c * K, K)], ids[b], dsems[b])

    for b in range(2):
        issue_src(b, b)
        issue_dst(b, b)
        pltpu.make_async_copy(src_hbm.at[pl.ds(0, K)], iss[b], isems[b]).wait()
        pltpu.async_copy(g_hbm.at[iss[b]], rows[b], gsems[b])

    def body(i, carry):
        c0 = i * 2
        for b in range(2):
            c = c0 + b
            cn = jnp.minimum(c + 2, NCH - 1)
            pltpu.make_async_copy(g_hbm.at[iss[b]], rows[b], gsems[b]).wait()
            issue_src(cn, b)
            pltpu.make_async_copy(dst_hbm.at[pl.ds(0, K)], ids[b], dsems[b]).wait()
            pltpu.sync_copy(rows[b], acc.at[ids[b]], add=True)
            issue_dst(cn, b)
            pltpu.make_async_copy(src_hbm.at[pl.ds(0, K)], iss[b], isems[b]).wait()
            pltpu.async_copy(g_hbm.at[iss[b]], rows[b], gsems[b])
        return carry

    lax.fori_loop(0, NCH // 2, body, 0)
    if NCH % 2:
        pltpu.make_async_copy(g_hbm.at[iss[0]], rows[0], gsems[0]).wait()
        pltpu.make_async_copy(dst_hbm.at[pl.ds(0, K)], ids[0], dsems[0]).wait()
        pltpu.sync_copy(rows[0], acc.at[ids[0]], add=True)
        pltpu.make_async_copy(g_hbm.at[iss[1]], rows[1], gsems[1]).wait()
        pltpu.make_async_copy(dst_hbm.at[pl.ds(0, K)], ids[1], dsems[1]).wait()
    else:
        for b in range(2):
            pltpu.make_async_copy(g_hbm.at[iss[b]], rows[b], gsems[b]).wait()
            pltpu.make_async_copy(dst_hbm.at[pl.ds(0, K)], ids[b], dsems[b]).wait()
    plsc.subcore_barrier()
    pltpu.sync_copy(acc.at[pl.ds(sid * RPS, RPS)],
                    out_hbm.at[cid, pl.ds(sid * RPS, RPS)])


_RB = 1000
_GRID = N // _RB


def _dis_of(degp):
    d = 1.0 + degp[0, :, 0:1] + degp[1, :, 0:1]
    return lax.rsqrt(d)


def _encode_body(x_ref, we_ref, be_ref, w1_ref, degp_ref, g1_ref):
    h = jnp.dot(x_ref[...], we_ref[...], preferred_element_type=jnp.float32)
    h = h + be_ref[...]
    dis = _dis_of(degp_ref[...])
    g1_ref[...] = dis * jnp.dot(h, w1_ref[...], preferred_element_type=jnp.float32)


def _encode_tc(x, W_enc, b_enc2, W1, degp):
    return pl.pallas_call(
        _encode_body,
        grid=(_GRID,),
        in_specs=[
            pl.BlockSpec((_RB, ATOM_DIM), lambda i: (i, 0)),
            pl.BlockSpec((ATOM_DIM, HIDDEN), lambda i: (0, 0)),
            pl.BlockSpec((1, HIDDEN), lambda i: (0, 0)),
            pl.BlockSpec((HIDDEN, HIDDEN), lambda i: (0, 0)),
            pl.BlockSpec((NC, _RB, DEGW), lambda i: (0, i, 0)),
        ],
        out_specs=pl.BlockSpec((_RB, HIDDEN), lambda i: (i, 0)),
        out_shape=jax.ShapeDtypeStruct((N, HIDDEN), jnp.float32),
    )(x, W_enc, b_enc2, W1, degp)


def _layer_body(p_ref, g_ref, degp_ref, b_ref, w_ref, out_ref):
    dis = _dis_of(degp_ref[...])
    agg = p_ref[0] + p_ref[1] + g_ref[...]
    h = jnp.maximum(dis * agg + b_ref[...], 0.0)
    out_ref[...] = dis * jnp.dot(h, w_ref[...], preferred_element_type=jnp.float32)


def _layer_tc(p, g, degp, b2, Wn):
    return pl.pallas_call(
        _layer_body,
        grid=(_GRID,),
        in_specs=[
            pl.BlockSpec((NC, _RB, HIDDEN), lambda i: (0, i, 0)),
            pl.BlockSpec((_RB, HIDDEN), lambda i: (i, 0)),
            pl.BlockSpec((NC, _RB, DEGW), lambda i: (0, i, 0)),
            pl.BlockSpec((1, HIDDEN), lambda i: (0, 0)),
            pl.BlockSpec((HIDDEN, HIDDEN), lambda i: (0, 0)),
        ],
        out_specs=pl.BlockSpec((_RB, HIDDEN), lambda i: (i, 0)),
        out_shape=jax.ShapeDtypeStruct((N, HIDDEN), jnp.float32),
    )(p, g, degp, b2, Wn)


def _pool_body(p_ref, g_ref, degp_ref, b_ref, batch_ref, sums_ref, cnt_ref):
    i = pl.program_id(0)
    dis = _dis_of(degp_ref[...])
    agg = p_ref[0] + p_ref[1] + g_ref[...]
    h = jnp.maximum(dis * agg + b_ref[...], 0.0)
    gids = lax.broadcasted_iota(jnp.int32, (NUM_GRAPHS, _RB), 0)
    onehot = (gids == batch_ref[0]).astype(jnp.float32)
    psum = jnp.dot(onehot, h, preferred_element_type=jnp.float32)
    pcnt = jnp.sum(onehot, axis=1, keepdims=True)
    pcnt = jnp.broadcast_to(pcnt, (NUM_GRAPHS, HIDDEN))

    @pl.when(i == 0)
    def _():
        sums_ref[...] = jnp.zeros_like(sums_ref)
        cnt_ref[...] = jnp.zeros_like(cnt_ref)

    sums_ref[...] += psum
    cnt_ref[...] += pcnt


def _pool_tc(p, g, degp, b2, batch3):
    return pl.pallas_call(
        _pool_body,
        grid=(_GRID,),
        in_specs=[
            pl.BlockSpec((NC, _RB, HIDDEN), lambda i: (0, i, 0)),
            pl.BlockSpec((_RB, HIDDEN), lambda i: (i, 0)),
            pl.BlockSpec((NC, _RB, DEGW), lambda i: (0, i, 0)),
            pl.BlockSpec((1, HIDDEN), lambda i: (0, 0)),
            pl.BlockSpec((1, 1, _RB), lambda i: (i, 0, 0)),
        ],
        out_specs=[
            pl.BlockSpec((NUM_GRAPHS, HIDDEN), lambda i: (0, 0)),
            pl.BlockSpec((NUM_GRAPHS, HIDDEN), lambda i: (0, 0)),
        ],
        out_shape=[
            jax.ShapeDtypeStruct((NUM_GRAPHS, HIDDEN), jnp.float32),
            jax.ShapeDtypeStruct((NUM_GRAPHS, HIDDEN), jnp.float32),
        ],
    )(p, g, degp, b2, batch3)


def _head_body(sums_ref, cnt_ref, wp_ref, bp_ref, lng_ref, lnb_ref, out_ref):
    mol = sums_ref[...] / jnp.maximum(cnt_ref[...], 1.0)
    y = jnp.dot(mol, wp_ref[...], preferred_element_type=jnp.float32)
    y = y + bp_ref[...]
    mu = jnp.mean(y, axis=1, keepdims=True)
    var = jnp.mean((y - mu) * (y - mu), axis=1, keepdims=True)
    y = (y - mu) * lax.rsqrt(var + 1e-5)
    out_ref[...] = y * lng_ref[...] + lnb_ref[...]


def _head_tc(sums, cnt, W_proj, bp2, lng2, lnb2):
    return pl.pallas_call(
        _head_body,
        out_shape=jax.ShapeDtypeStruct((NUM_GRAPHS, NODE_DIM), jnp.float32),
    )(sums, cnt, W_proj, bp2, lng2, lnb2)


def kernel(x, edge_index, batch, W_enc, b_enc, W_convs, b_convs,
           W_proj, b_proj, ln_g, ln_b):
    src = edge_index[0]
    dst = edge_index[1]
    dst3 = dst.reshape(NW, NCH, K)
    batch3 = batch.reshape(_GRID, 1, _RB)
    zeros_h = jnp.zeros((NPAD, HIDDEN), jnp.float32)
    ones_k = jnp.ones((K, DEGW), jnp.float32)
    b_enc2 = b_enc.reshape(1, HIDDEN)
    bc2 = [b_convs[i].reshape(1, HIDDEN) for i in range(3)]

    degp = _deg_sc(dst3, ones_k, zeros_h)
    g = _encode_tc(x, W_enc, b_enc2, W_convs[0], degp)
    p = _agg_sc(g, src, dst, zeros_h)
    g = _layer_tc(p, g, degp, bc2[0], W_convs[1])
    p = _agg_sc(g, src, dst, zeros_h)
    g = _layer_tc(p, g, degp, bc2[1], W_convs[2])
    p = _agg_sc(g, src, dst, zeros_h)
    sums, cnt = _pool_tc(p, g, degp, bc2[2], batch3)
    return _head_tc(sums, cnt, W_proj, b_proj.reshape(1, NODE_DIM),
                    ln_g.reshape(1, NODE_DIM), ln_b.reshape(1, NODE_DIM))
